# Initial kernel scaffold; baseline (speedup 1.0000x reference)
import jax
import jax.numpy as jnp
from jax.experimental import pallas as pl


def kernel(x, edge_index, edge_weight, W_z, b_z, W_r, b_r, W_h, b_h, L_z_w, L_z_b, L_r_w, L_r_b, L_h_w, L_h_b, W_out, b_out):
    raise NotImplementedError("write your pallas kernel here")


# speedup vs baseline: 34.2678x; 34.2678x over previous
import functools

import jax
import jax.numpy as jnp
from jax import lax
from jax.experimental import pallas as pl
from jax.experimental.pallas import tpu as pltpu
from jax.experimental.pallas import tpu_sc as plsc

N = 10000
E = 320000
IN_C = 128
HID = 256
OUT_C = 128

NC = 2
NS = 16
L = 16
NW = NC * NS

EPW = E // NW
G = 128
NBLK = E // G
BLK_PER_W = -(-NBLK // NW)
NPAD = 10240
ROWS_PER_TILE = NPAD // NS
ZROWS = 128

_mesh = plsc.VectorSubcoreMesh(core_axis_name="c", subcore_axis_name="s")


@functools.partial(
    pl.kernel,
    out_type=jax.ShapeDtypeStruct((NW, N), jnp.float32),
    mesh=_mesh,
    scratch_types=[
        pltpu.VMEM((N,), jnp.float32),
        pltpu.VMEM((EPW,), jnp.int32),
        pltpu.VMEM((EPW,), jnp.float32),
    ],
    compiler_params=pltpu.CompilerParams(needs_layout_passes=False),
)
def _deg_kernel(dst_hbm, w_hbm, out_hbm, deg_v, dst_v, w_v):
    cid = lax.axis_index("c")
    sid = lax.axis_index("s")
    wid = sid * NC + cid
    base = wid * EPW

    zeros = jnp.zeros((L,), jnp.float32)

    def zbody(i, _):
        deg_v[pl.ds(pl.multiple_of(i * L, L), L)] = zeros
        return ()

    lax.fori_loop(0, N // L, zbody, ())

    pltpu.sync_copy(dst_hbm.at[pl.ds(base, EPW)], dst_v)
    pltpu.sync_copy(w_hbm.at[pl.ds(base, EPW)], w_v)

    def body(i, _):
        off = pl.multiple_of(i * L, L)
        idx = dst_v[pl.ds(off, L)]
        vals = w_v[pl.ds(off, L)]
        plsc.addupdate_scatter(deg_v, [idx], vals)
        return ()

    lax.fori_loop(0, EPW // L, body, ())

    pltpu.sync_copy(deg_v, out_hbm.at[wid])


def _dinv_body(pdeg_ref, dinv_ref, self_ref):
    deg = jnp.sum(pdeg_ref[...], axis=0, keepdims=True)
    dinv = lax.rsqrt(deg + 1.0)
    dinv_ref[...] = dinv
    self_ref[...] = dinv * dinv


_dinv_call = pl.pallas_call(
    _dinv_body,
    out_shape=[
        jax.ShapeDtypeStruct((1, N), jnp.float32),
        jax.ShapeDtypeStruct((1, N), jnp.float32),
    ],
)


@functools.partial(
    pl.kernel,
    out_type=jax.ShapeDtypeStruct((NC, NPAD, IN_C), jnp.float32),
    mesh=_mesh,
    scratch_types=[
        pltpu.VMEM((N,), jnp.float32),
        pltpu.VMEM((G,), jnp.int32),
        pltpu.VMEM((G,), jnp.int32),
        pltpu.VMEM((G,), jnp.float32),
        pltpu.VMEM((G, IN_C), jnp.float32),
        pltpu.VMEM((ZROWS, IN_C), jnp.float32),
        pltpu.VMEM_SHARED((NPAD, IN_C), jnp.float32),
        pltpu.SemaphoreType.DMA,
    ],
    compiler_params=pltpu.CompilerParams(needs_layout_passes=False),
)
def _agg_kernel(src_hbm, dst_hbm, w_hbm, dinv_hbm, x_hbm, out_hbm,
                dinv_v, src_v, dst_v, w_v, rows_v, zero_v, agg_sh,
                sem):
    cid = lax.axis_index("c")
    sid = lax.axis_index("s")
    wid = sid * NC + cid

    pltpu.sync_copy(dinv_hbm, dinv_v)

    zeros = jnp.zeros((L,), jnp.float32)

    def zbody(i, _):
        r = i // (IN_C // L)
        coff = pl.multiple_of((i % (IN_C // L)) * L, L)
        zero_v[r, pl.ds(coff, L)] = zeros
        return ()

    lax.fori_loop(0, ZROWS * (IN_C // L), zbody, ())
    for p in range(ROWS_PER_TILE // ZROWS):
        pltpu.sync_copy(zero_v,
                        agg_sh.at[pl.ds(sid * ROWS_PER_TILE + p * ZROWS,
                                        ZROWS)])
    plsc.subcore_barrier()

    def blk_body(k, _):
        b = wid + NW * k

        @pl.when(b < NBLK)
        def _():
            base = b * G
            pltpu.sync_copy(src_hbm.at[pl.ds(base, G)], src_v)
            pltpu.sync_copy(dst_hbm.at[pl.ds(base, G)], dst_v)
            pltpu.sync_copy(w_hbm.at[pl.ds(base, G)], w_v)
            pltpu.async_copy(x_hbm.at[src_v], rows_v, sem).wait()

            def nbody(i, _):
                off = pl.multiple_of(i * L, L)
                si = src_v[pl.ds(off, L)]
                dsv = plsc.load_gather(dinv_v, [si])
                nv = dsv * w_v[pl.ds(off, L)]
                for j in range(L):
                    ne = nv[j]
                    row = i * L + j
                    for c in range(IN_C // L):
                        rows_v[row, pl.ds(c * L, L)] = (
                            rows_v[row, pl.ds(c * L, L)] * ne)
                return ()

            lax.fori_loop(0, G // L, nbody, ())

            pltpu.sync_copy(rows_v, agg_sh.at[dst_v], add=True)

        return ()

    lax.fori_loop(0, BLK_PER_W, blk_body, ())

    plsc.subcore_barrier()
    pltpu.sync_copy(agg_sh.at[pl.ds(sid * ROWS_PER_TILE, ROWS_PER_TILE)],
                    out_hbm.at[cid, pl.ds(sid * ROWS_PER_TILE,
                                          ROWS_PER_TILE)])


def _fold_body(wz_ref, lz_ref, bz_ref, lzb_ref, wh_ref, lh_ref, bh_ref,
               lhb_ref, mz_ref, cz_ref, mh_ref, ch_ref):
    hi = lax.Precision.HIGHEST
    mz_ref[...] = jnp.dot(wz_ref[...], lz_ref[...], precision=hi)
    cz_ref[...] = jnp.dot(bz_ref[...], lz_ref[...], precision=hi) + lzb_ref[...]
    mh_ref[...] = jnp.dot(wh_ref[...], lh_ref[...], precision=hi)
    ch_ref[...] = jnp.dot(bh_ref[...], lh_ref[...], precision=hi) + lhb_ref[...]


_fold_call = pl.pallas_call(
    _fold_body,
    out_shape=[
        jax.ShapeDtypeStruct((IN_C, HID), jnp.float32),
        jax.ShapeDtypeStruct((1, HID), jnp.float32),
        jax.ShapeDtypeStruct((IN_C, HID), jnp.float32),
        jax.ShapeDtypeStruct((1, HID), jnp.float32),
    ],
)


_BN = 1000


def _dense_body(x_ref, a0_ref, a1_ref, dv_ref, st_ref, mz_ref, cz_ref,
                mh_ref, ch_ref, wo_ref, bo_ref, out_ref):
    hi = lax.Precision.HIGHEST
    a = dv_ref[...] * (a0_ref[...] + a1_ref[...]) + st_ref[...] * x_ref[...]
    az = jnp.dot(a, mz_ref[...], precision=hi) + cz_ref[...]
    ah = jnp.dot(a, mh_ref[...], precision=hi) + ch_ref[...]
    hn = (1.0 - jax.nn.sigmoid(az)) * jnp.tanh(ah)
    out_ref[...] = jnp.dot(hn, wo_ref[...], precision=hi) + bo_ref[...]


_dense_call = pl.pallas_call(
    _dense_body,
    grid=(N // _BN,),
    in_specs=[
        pl.BlockSpec((_BN, IN_C), lambda i: (i, 0)),
        pl.BlockSpec((_BN, IN_C), lambda i: (i, 0)),
        pl.BlockSpec((_BN, IN_C), lambda i: (i, 0)),
        pl.BlockSpec((_BN, 1), lambda i: (i, 0)),
        pl.BlockSpec((_BN, 1), lambda i: (i, 0)),
        pl.BlockSpec((IN_C, HID), lambda i: (0, 0)),
        pl.BlockSpec((1, HID), lambda i: (0, 0)),
        pl.BlockSpec((IN_C, HID), lambda i: (0, 0)),
        pl.BlockSpec((1, HID), lambda i: (0, 0)),
        pl.BlockSpec((HID, OUT_C), lambda i: (0, 0)),
        pl.BlockSpec((1, OUT_C), lambda i: (0, 0)),
    ],
    out_specs=pl.BlockSpec((_BN, OUT_C), lambda i: (i, 0)),
    out_shape=jax.ShapeDtypeStruct((N, OUT_C), jnp.float32),
)


def kernel(x, edge_index, edge_weight, W_z, b_z, W_r, b_r, W_h, b_h,
           L_z_w, L_z_b, L_r_w, L_r_b, L_h_w, L_h_b, W_out, b_out):
    src = edge_index[0]
    dst = edge_index[1]

    pdeg = _deg_kernel(dst, edge_weight)
    dinv_row, selfterm_row = _dinv_call(pdeg)
    dinv = dinv_row.reshape(N)
    dinv_col = dinv_row.reshape(N, 1)
    selfterm = selfterm_row.reshape(N, 1)

    agg = _agg_kernel(src, dst, edge_weight, dinv, x)

    mz, cz, mh, ch = _fold_call(W_z, L_z_w[:HID], b_z.reshape(1, HID),
                                L_z_b.reshape(1, HID), W_h, L_h_w[:HID],
                                b_h.reshape(1, HID), L_h_b.reshape(1, HID))

    return _dense_call(x, agg[0], agg[1], dinv_col, selfterm, mz, cz, mh,
                       ch, W_out, b_out.reshape(1, OUT_C))
